# Initial kernel scaffold; baseline (speedup 1.0000x reference)
import jax
import jax.numpy as jnp
from jax.experimental import pallas as pl


def kernel(x_src, x_dst, edge_index, W_src, b_src, W_dst, b_dst, W1_l, b1_l, W1_r, W2_l, b2_l, W2_r, gamma, beta):
    raise NotImplementedError("write your pallas kernel here")


# speedup vs baseline: 3.9731x; 3.9731x over previous
import functools

import jax
import jax.numpy as jnp
from jax import lax
from jax.experimental import pallas as pl
from jax.experimental.pallas import tpu as pltpu
from jax.experimental.pallas import tpu_sc as plsc

N_NODES = 10000
N_EDGES = 320000
FD = 128

NC, NS = 2, 16
NW = NC * NS
EPW = N_EDGES // NW
CHUNK = 80
NCHUNK = EPW // CHUNK
N_PAD = 10240
RPT = N_PAD // NS
DR = 64
NDR = RPT // DR


def _mk_mesh():
    return plsc.VectorSubcoreMesh(
        core_axis_name="c", subcore_axis_name="s", num_cores=NC, num_subcores=NS
    )


def _make_seg_sum():
    @functools.partial(
        pl.kernel,
        out_type=jax.ShapeDtypeStruct((NC, N_PAD, FD), jnp.float32),
        mesh=_mk_mesh(),
        scratch_types=[
            pltpu.VMEM((CHUNK,), jnp.int32),
            pltpu.VMEM((CHUNK,), jnp.int32),
            pltpu.VMEM((CHUNK, FD), jnp.float32),
            pltpu.VMEM((DR, FD), jnp.float32),
            pltpu.VMEM_SHARED((N_PAD, FD), jnp.float32),
            pltpu.SemaphoreType.DMA,
        ],
    )
    def k(table, src, dst, out, src_v, dst_v, rows_v, stage_v, acc, sem):
        c = lax.axis_index("c")
        s = lax.axis_index("s")
        wid = c * NS + s
        zero16 = jnp.zeros((16,), jnp.float32)

        def zrow(i, _):
            r = i // (FD // 16)
            q = i % (FD // 16)
            stage_v[r, pl.ds(q * 16, 16)] = zero16
            return 0
        lax.fori_loop(0, DR * (FD // 16), zrow, 0)

        row0 = s * RPT
        for kk in range(NDR):
            pltpu.sync_copy(stage_v, acc.at[pl.ds(row0 + kk * DR, DR)])
        plsc.subcore_barrier()

        ebase = wid * EPW

        def step(g, _):
            off = pl.multiple_of(ebase + g * CHUNK, 8)
            pltpu.sync_copy(src.at[pl.ds(off, CHUNK)], src_v)
            pltpu.sync_copy(dst.at[pl.ds(off, CHUNK)], dst_v)
            pltpu.async_copy(table.at[src_v], rows_v, sem).wait()
            pltpu.sync_copy(rows_v, acc.at[dst_v], add=True)
            return 0
        lax.fori_loop(0, NCHUNK, step, 0)
        plsc.subcore_barrier()

        for kk in range(NDR):
            r = row0 + kk * DR
            pltpu.sync_copy(acc.at[pl.ds(r, DR)], stage_v)
            pltpu.sync_copy(stage_v, out.at[c, pl.ds(r, DR)])

    return k


def _make_count():
    @functools.partial(
        pl.kernel,
        out_type=jax.ShapeDtypeStruct((NC, N_PAD, FD), jnp.float32),
        mesh=_mk_mesh(),
        scratch_types=[
            pltpu.VMEM((CHUNK,), jnp.int32),
            pltpu.VMEM((CHUNK, FD), jnp.float32),
            pltpu.VMEM((DR, FD), jnp.float32),
            pltpu.VMEM_SHARED((N_PAD, FD), jnp.float32),
        ],
    )
    def k(dst, out, dst_v, ones_v, stage_v, acc):
        c = lax.axis_index("c")
        s = lax.axis_index("s")
        wid = c * NS + s
        zero16 = jnp.zeros((16,), jnp.float32)
        one16 = jnp.ones((16,), jnp.float32)

        def zrow(i, _):
            r = i // (FD // 16)
            q = i % (FD // 16)
            stage_v[r, pl.ds(q * 16, 16)] = zero16
            return 0
        lax.fori_loop(0, DR * (FD // 16), zrow, 0)

        def orow(i, _):
            r = i // (FD // 16)
            q = i % (FD // 16)
            ones_v[r, pl.ds(q * 16, 16)] = one16
            return 0
        lax.fori_loop(0, CHUNK * (FD // 16), orow, 0)

        row0 = s * RPT
        for kk in range(NDR):
            pltpu.sync_copy(stage_v, acc.at[pl.ds(row0 + kk * DR, DR)])
        plsc.subcore_barrier()

        ebase = wid * EPW

        def step(g, _):
            off = pl.multiple_of(ebase + g * CHUNK, 8)
            pltpu.sync_copy(dst.at[pl.ds(off, CHUNK)], dst_v)
            pltpu.sync_copy(ones_v, acc.at[dst_v], add=True)
            return 0
        lax.fori_loop(0, NCHUNK, step, 0)
        plsc.subcore_barrier()

        for kk in range(NDR):
            r = row0 + kk * DR
            pltpu.sync_copy(acc.at[pl.ds(r, DR)], stage_v)
            pltpu.sync_copy(stage_v, out.at[c, pl.ds(r, DR)])

    return k


_seg_sum = _make_seg_sum()
_count_seg = _make_count()


HALF = N_NODES // 2


def _project(x_src, x_dst, W_src, b_src, W_dst, b_dst):
    def body(xs, xd, ws, bs, wd, bd, o):
        o[0:HALF, :] = (
            jnp.dot(xs[...], ws[...], preferred_element_type=jnp.float32) + bs[...]
        )
        o[HALF:N_NODES, :] = (
            jnp.dot(xd[...], wd[...], preferred_element_type=jnp.float32) + bd[...]
        )

    return pl.pallas_call(
        body,
        out_shape=jax.ShapeDtypeStruct((N_NODES, FD), jnp.float32),
    )(x_src, x_dst, W_src, b_src.reshape(1, FD), W_dst, b_dst.reshape(1, FD))


RB = 400


def _layer1(parts, cnts, x, W1_l, b1_l, W1_r):
    grid = (N_NODES // RB,)

    def body(p_ref, c_ref, x_ref, wl, bl, wr, hpre_ref, stat_ref):
        i = pl.program_id(0)
        psum = p_ref[0] + p_ref[1]
        cnt = c_ref[0, :, 0:1] + c_ref[1, :, 0:1]
        inv = 1.0 / jnp.maximum(cnt, 1.0)
        agg = psum * inv
        hp = (
            jnp.dot(agg, wl[...], preferred_element_type=jnp.float32)
            + bl[...]
            + jnp.dot(x_ref[...], wr[...], preferred_element_type=jnp.float32)
        )
        hpre_ref[...] = hp

        @pl.when(i == 0)
        def _():
            stat_ref[...] = jnp.zeros_like(stat_ref)

        stat_ref[0:1, :] += jnp.sum(hp, axis=0, keepdims=True)
        stat_ref[1:2, :] += jnp.sum(hp * hp, axis=0, keepdims=True)

    return pl.pallas_call(
        body,
        grid=grid,
        in_specs=[
            pl.BlockSpec((2, RB, FD), lambda i: (0, i, 0)),
            pl.BlockSpec((2, RB, FD), lambda i: (0, i, 0)),
            pl.BlockSpec((RB, FD), lambda i: (i, 0)),
            pl.BlockSpec((FD, FD), lambda i: (0, 0)),
            pl.BlockSpec((1, FD), lambda i: (0, 0)),
            pl.BlockSpec((FD, FD), lambda i: (0, 0)),
        ],
        out_specs=[
            pl.BlockSpec((RB, FD), lambda i: (i, 0)),
            pl.BlockSpec((8, FD), lambda i: (0, 0)),
        ],
        out_shape=[
            jax.ShapeDtypeStruct((N_NODES, FD), jnp.float32),
            jax.ShapeDtypeStruct((8, FD), jnp.float32),
        ],
    )(parts, cnts, x, W1_l, b1_l.reshape(1, FD), W1_r)


def _bn_relu(hpre, stats, gamma, beta):
    grid = (N_NODES // RB,)

    def body(h_ref, st_ref, g_ref, b_ref, o_ref):
        mu = st_ref[0:1, :] / N_NODES
        var = st_ref[1:2, :] / N_NODES - mu * mu
        scale = g_ref[...] * lax.rsqrt(var + 1e-5)
        shift = b_ref[...] - mu * scale
        o_ref[...] = jnp.maximum(h_ref[...] * scale + shift, 0.0)

    return pl.pallas_call(
        body,
        grid=grid,
        in_specs=[
            pl.BlockSpec((RB, FD), lambda i: (i, 0)),
            pl.BlockSpec((8, FD), lambda i: (0, 0)),
            pl.BlockSpec((1, FD), lambda i: (0, 0)),
            pl.BlockSpec((1, FD), lambda i: (0, 0)),
        ],
        out_specs=pl.BlockSpec((RB, FD), lambda i: (i, 0)),
        out_shape=jax.ShapeDtypeStruct((N_NODES, FD), jnp.float32),
    )(hpre, stats, gamma.reshape(1, FD), beta.reshape(1, FD))


def _layer2(parts, cnts, h, W2_l, b2_l, W2_r):
    grid = (N_NODES // RB,)

    def body(p_ref, c_ref, h_ref, wl, bl, wr, o_ref):
        psum = p_ref[0] + p_ref[1]
        cnt = c_ref[0, :, 0:1] + c_ref[1, :, 0:1]
        inv = 1.0 / jnp.maximum(cnt, 1.0)
        agg = psum * inv
        o_ref[...] = (
            jnp.dot(agg, wl[...], preferred_element_type=jnp.float32)
            + bl[...]
            + jnp.dot(h_ref[...], wr[...], preferred_element_type=jnp.float32)
        )

    return pl.pallas_call(
        body,
        grid=grid,
        in_specs=[
            pl.BlockSpec((2, RB, FD), lambda i: (0, i, 0)),
            pl.BlockSpec((2, RB, FD), lambda i: (0, i, 0)),
            pl.BlockSpec((RB, FD), lambda i: (i, 0)),
            pl.BlockSpec((FD, FD), lambda i: (0, 0)),
            pl.BlockSpec((1, FD), lambda i: (0, 0)),
            pl.BlockSpec((FD, FD), lambda i: (0, 0)),
        ],
        out_specs=pl.BlockSpec((RB, FD), lambda i: (i, 0)),
        out_shape=jax.ShapeDtypeStruct((N_NODES, FD), jnp.float32),
    )(parts, cnts, h, W2_l, b2_l.reshape(1, FD), W2_r)


def kernel(x_src, x_dst, edge_index, W_src, b_src, W_dst, b_dst,
           W1_l, b1_l, W1_r, W2_l, b2_l, W2_r, gamma, beta):
    src = edge_index[0]
    dst = edge_index[1]
    x = _project(x_src, x_dst, W_src, b_src, W_dst, b_dst)
    cnts = _count_seg(dst)
    parts1 = _seg_sum(x, src, dst)
    hpre, stats = _layer1(parts1, cnts, x, W1_l, b1_l, W1_r)
    h = _bn_relu(hpre, stats, gamma, beta)
    parts2 = _seg_sum(h, src, dst)
    out = _layer2(parts2, cnts, h, W2_l, b2_l, W2_r)
    return out


# speedup vs baseline: 8.0215x; 2.0189x over previous
import functools

import jax
import jax.numpy as jnp
from jax import lax
from jax.experimental import pallas as pl
from jax.experimental.pallas import tpu as pltpu
from jax.experimental.pallas import tpu_sc as plsc

N_NODES = 10000
N_EDGES = 320000
FD = 128

NC, NS = 2, 16
NW = NC * NS
EPW = N_EDGES // NW
CHUNK = 80
NCHUNK = EPW // CHUNK
N_PAD = 10240
RPT = N_PAD // NS
DR = 32
NDR = RPT // DR
NQ = (NCHUNK + 3) // 4


def _mk_mesh():
    return plsc.VectorSubcoreMesh(
        core_axis_name="c", subcore_axis_name="s", num_cores=NC, num_subcores=NS
    )


def _make_seg_sum():
    @functools.partial(
        pl.kernel,
        out_type=jax.ShapeDtypeStruct((NC, N_PAD, FD), jnp.float32),
        mesh=_mk_mesh(),
        scratch_types=[
            [pltpu.VMEM((CHUNK,), jnp.int32) for _ in range(4)],
            [pltpu.VMEM((CHUNK,), jnp.int32) for _ in range(4)],
            [pltpu.VMEM((CHUNK, FD), jnp.float32) for _ in range(2)],
            pltpu.VMEM((DR, FD), jnp.float32),
            pltpu.VMEM_SHARED((N_PAD, FD), jnp.float32),
            pltpu.SemaphoreType.DMA,
            pltpu.SemaphoreType.DMA,
        ],
    )
    def k(table, src3, dst3, out, srcs, dsts, rows, stage_v, acc, semg, semi):
        c = lax.axis_index("c")
        s = lax.axis_index("s")
        wid = c * NS + s
        zero16 = jnp.zeros((16,), jnp.float32)

        def zrow(i, _):
            r = i // (FD // 16)
            q = i % (FD // 16)
            stage_v[r, pl.ds(q * 16, 16)] = zero16
            return 0
        lax.fori_loop(0, DR * (FD // 16), zrow, 0)

        row0 = s * RPT

        def zcp(kk, _):
            r = pl.multiple_of(row0 + kk * DR, 8)
            pltpu.sync_copy(stage_v, acc.at[pl.ds(r, DR)])
            return 0
        lax.fori_loop(0, NDR, zcp, 0)
        plsc.subcore_barrier()

        pltpu.sync_copy(src3.at[wid, 0], srcs[0])
        pltpu.sync_copy(dst3.at[wid, 0], dsts[0])
        pltpu.sync_copy(src3.at[wid, 1], srcs[1])
        pltpu.sync_copy(dst3.at[wid, 1], dsts[1])
        pltpu.async_copy(table.at[srcs[0]], rows[0], semg)
        pltpu.async_copy(table.at[srcs[1]], rows[1], semg)
        pltpu.async_copy(src3.at[wid, 2], srcs[2], semi)
        pltpu.async_copy(dst3.at[wid, 2], dsts[2], semi)
        pltpu.async_copy(src3.at[wid, 3], srcs[3], semi)
        pltpu.async_copy(dst3.at[wid, 3], dsts[3], semi)

        def quad(qq, _):
            q = qq * 4
            for j in range(4):
                ch = q + j
                rb = rows[j % 2]
                ib = j

                @pl.when(ch < NCHUNK)
                def _():
                    pltpu.make_async_copy(table.at[pl.ds(0, CHUNK)], rb, semg).wait()
                    pltpu.sync_copy(rb, acc.at[dsts[ib]], add=True)

                @pl.when(ch + 4 < NCHUNK)
                def _():
                    pltpu.async_copy(src3.at[wid, ch + 4], srcs[ib], semi)
                    pltpu.async_copy(dst3.at[wid, ch + 4], dsts[ib], semi)

                @pl.when(ch + 2 < NCHUNK)
                def _():
                    ib2 = (j + 2) % 4
                    pltpu.make_async_copy(src3.at[wid, 0], srcs[ib2], semi).wait()
                    pltpu.make_async_copy(dst3.at[wid, 0], dsts[ib2], semi).wait()
                    pltpu.async_copy(table.at[srcs[ib2]], rb, semg)
            return 0
        lax.fori_loop(0, NQ, quad, 0)
        plsc.subcore_barrier()

        def drain(kk, _):
            r = pl.multiple_of(row0 + kk * DR, 8)
            pltpu.sync_copy(acc.at[pl.ds(r, DR)], stage_v)
            pltpu.sync_copy(stage_v, out.at[c, pl.ds(r, DR)])
            return 0
        lax.fori_loop(0, NDR, drain, 0)

    return k


def _make_count():
    @functools.partial(
        pl.kernel,
        out_type=jax.ShapeDtypeStruct((NC, N_PAD, FD), jnp.float32),
        mesh=_mk_mesh(),
        scratch_types=[
            [pltpu.VMEM((CHUNK,), jnp.int32) for _ in range(4)],
            pltpu.VMEM((CHUNK, FD), jnp.float32),
            pltpu.VMEM((DR, FD), jnp.float32),
            pltpu.VMEM_SHARED((N_PAD, FD), jnp.float32),
            pltpu.SemaphoreType.DMA,
        ],
    )
    def k(dst3, out, dsts, ones_v, stage_v, acc, semi):
        c = lax.axis_index("c")
        s = lax.axis_index("s")
        wid = c * NS + s
        zero16 = jnp.zeros((16,), jnp.float32)
        one16 = jnp.ones((16,), jnp.float32)

        def zrow(i, _):
            r = i // (FD // 16)
            q = i % (FD // 16)
            stage_v[r, pl.ds(q * 16, 16)] = zero16
            return 0
        lax.fori_loop(0, DR * (FD // 16), zrow, 0)

        def orow(i, _):
            r = i // (FD // 16)
            q = i % (FD // 16)
            ones_v[r, pl.ds(q * 16, 16)] = one16
            return 0
        lax.fori_loop(0, CHUNK * (FD // 16), orow, 0)

        row0 = s * RPT

        def zcp(kk, _):
            r = pl.multiple_of(row0 + kk * DR, 8)
            pltpu.sync_copy(stage_v, acc.at[pl.ds(r, DR)])
            return 0
        lax.fori_loop(0, NDR, zcp, 0)
        plsc.subcore_barrier()

        for j in range(4):
            pltpu.async_copy(dst3.at[wid, j], dsts[j], semi)

        def quad(qq, _):
            q = qq * 4
            for j in range(4):
                ch = q + j

                @pl.when(ch < NCHUNK)
                def _():
                    pltpu.make_async_copy(dst3.at[wid, 0], dsts[j], semi).wait()
                    pltpu.sync_copy(ones_v, acc.at[dsts[j]], add=True)

                @pl.when(ch + 4 < NCHUNK)
                def _():
                    pltpu.async_copy(dst3.at[wid, ch + 4], dsts[j], semi)
            return 0
        lax.fori_loop(0, NQ, quad, 0)
        plsc.subcore_barrier()

        def drain(kk, _):
            r = pl.multiple_of(row0 + kk * DR, 8)
            pltpu.sync_copy(acc.at[pl.ds(r, DR)], stage_v)
            pltpu.sync_copy(stage_v, out.at[c, pl.ds(r, DR)])
            return 0
        lax.fori_loop(0, NDR, drain, 0)

    return k


_seg_sum = _make_seg_sum()
_count_seg = _make_count()


HALF = N_NODES // 2


def _project(x_src, x_dst, W_src, b_src, W_dst, b_dst):
    def body(xs, xd, ws, bs, wd, bd, o):
        o[0:HALF, :] = (
            jnp.dot(xs[...], ws[...], preferred_element_type=jnp.float32) + bs[...]
        )
        o[HALF:N_NODES, :] = (
            jnp.dot(xd[...], wd[...], preferred_element_type=jnp.float32) + bd[...]
        )

    return pl.pallas_call(
        body,
        out_shape=jax.ShapeDtypeStruct((N_NODES, FD), jnp.float32),
    )(x_src, x_dst, W_src, b_src.reshape(1, FD), W_dst, b_dst.reshape(1, FD))


RB = 400


def _layer1(parts, cnts, x, W1_l, b1_l, W1_r):
    grid = (N_NODES // RB,)

    def body(p_ref, c_ref, x_ref, wl, bl, wr, hpre_ref, stat_ref):
        i = pl.program_id(0)
        psum = p_ref[0] + p_ref[1]
        cnt = c_ref[0, :, 0:1] + c_ref[1, :, 0:1]
        inv = 1.0 / jnp.maximum(cnt, 1.0)
        agg = psum * inv
        hp = (
            jnp.dot(agg, wl[...], preferred_element_type=jnp.float32)
            + bl[...]
            + jnp.dot(x_ref[...], wr[...], preferred_element_type=jnp.float32)
        )
        hpre_ref[...] = hp

        @pl.when(i == 0)
        def _():
            stat_ref[...] = jnp.zeros_like(stat_ref)

        stat_ref[0:1, :] += jnp.sum(hp, axis=0, keepdims=True)
        stat_ref[1:2, :] += jnp.sum(hp * hp, axis=0, keepdims=True)

    return pl.pallas_call(
        body,
        grid=grid,
        in_specs=[
            pl.BlockSpec((2, RB, FD), lambda i: (0, i, 0)),
            pl.BlockSpec((2, RB, FD), lambda i: (0, i, 0)),
            pl.BlockSpec((RB, FD), lambda i: (i, 0)),
            pl.BlockSpec((FD, FD), lambda i: (0, 0)),
            pl.BlockSpec((1, FD), lambda i: (0, 0)),
            pl.BlockSpec((FD, FD), lambda i: (0, 0)),
        ],
        out_specs=[
            pl.BlockSpec((RB, FD), lambda i: (i, 0)),
            pl.BlockSpec((8, FD), lambda i: (0, 0)),
        ],
        out_shape=[
            jax.ShapeDtypeStruct((N_NODES, FD), jnp.float32),
            jax.ShapeDtypeStruct((8, FD), jnp.float32),
        ],
    )(parts, cnts, x, W1_l, b1_l.reshape(1, FD), W1_r)


def _bn_relu(hpre, stats, gamma, beta):
    grid = (N_NODES // RB,)

    def body(h_ref, st_ref, g_ref, b_ref, o_ref):
        mu = st_ref[0:1, :] / N_NODES
        var = st_ref[1:2, :] / N_NODES - mu * mu
        scale = g_ref[...] * lax.rsqrt(var + 1e-5)
        shift = b_ref[...] - mu * scale
        o_ref[...] = jnp.maximum(h_ref[...] * scale + shift, 0.0)

    return pl.pallas_call(
        body,
        grid=grid,
        in_specs=[
            pl.BlockSpec((RB, FD), lambda i: (i, 0)),
            pl.BlockSpec((8, FD), lambda i: (0, 0)),
            pl.BlockSpec((1, FD), lambda i: (0, 0)),
            pl.BlockSpec((1, FD), lambda i: (0, 0)),
        ],
        out_specs=pl.BlockSpec((RB, FD), lambda i: (i, 0)),
        out_shape=jax.ShapeDtypeStruct((N_NODES, FD), jnp.float32),
    )(hpre, stats, gamma.reshape(1, FD), beta.reshape(1, FD))


def _layer2(parts, cnts, h, W2_l, b2_l, W2_r):
    grid = (N_NODES // RB,)

    def body(p_ref, c_ref, h_ref, wl, bl, wr, o_ref):
        psum = p_ref[0] + p_ref[1]
        cnt = c_ref[0, :, 0:1] + c_ref[1, :, 0:1]
        inv = 1.0 / jnp.maximum(cnt, 1.0)
        agg = psum * inv
        o_ref[...] = (
            jnp.dot(agg, wl[...], preferred_element_type=jnp.float32)
            + bl[...]
            + jnp.dot(h_ref[...], wr[...], preferred_element_type=jnp.float32)
        )

    return pl.pallas_call(
        body,
        grid=grid,
        in_specs=[
            pl.BlockSpec((2, RB, FD), lambda i: (0, i, 0)),
            pl.BlockSpec((2, RB, FD), lambda i: (0, i, 0)),
            pl.BlockSpec((RB, FD), lambda i: (i, 0)),
            pl.BlockSpec((FD, FD), lambda i: (0, 0)),
            pl.BlockSpec((1, FD), lambda i: (0, 0)),
            pl.BlockSpec((FD, FD), lambda i: (0, 0)),
        ],
        out_specs=pl.BlockSpec((RB, FD), lambda i: (i, 0)),
        out_shape=jax.ShapeDtypeStruct((N_NODES, FD), jnp.float32),
    )(parts, cnts, h, W2_l, b2_l.reshape(1, FD), W2_r)


def kernel(x_src, x_dst, edge_index, W_src, b_src, W_dst, b_dst,
           W1_l, b1_l, W1_r, W2_l, b2_l, W2_r, gamma, beta):
    src = edge_index[0].reshape(NW, NCHUNK, CHUNK)
    dst = edge_index[1].reshape(NW, NCHUNK, CHUNK)
    x = _project(x_src, x_dst, W_src, b_src, W_dst, b_dst)
    cnts = _count_seg(dst)
    parts1 = _seg_sum(x, src, dst)
    hpre, stats = _layer1(parts1, cnts, x, W1_l, b1_l, W1_r)
    h = _bn_relu(hpre, stats, gamma, beta)
    parts2 = _seg_sum(h, src, dst)
    out = _layer2(parts2, cnts, h, W2_l, b2_l, W2_r)
    return out


# speedup vs baseline: 8.2746x; 1.0316x over previous
import functools

import jax
import jax.numpy as jnp
from jax import lax
from jax.experimental import pallas as pl
from jax.experimental.pallas import tpu as pltpu
from jax.experimental.pallas import tpu_sc as plsc

N_NODES = 10000
N_EDGES = 320000
FD = 128

NC, NS = 2, 16
NW = NC * NS
EPW = N_EDGES // NW
CHUNK = 80
NCHUNK = EPW // CHUNK
N_PAD = 10240
RPT = N_PAD // NS
DR = 64
NDR = RPT // DR
NQ = (NCHUNK + 3) // 4


def _mk_mesh():
    return plsc.VectorSubcoreMesh(
        core_axis_name="c", subcore_axis_name="s", num_cores=NC, num_subcores=NS
    )


def _drain_acc(c, row0, acc, out, stages, semr, semw):
    def rd(kk, b):
        r = pl.multiple_of(row0 + kk * DR, 8)
        pltpu.async_copy(acc.at[pl.ds(r, DR)], stages[b], semr)

    def wr(kk, b):
        r = pl.multiple_of(row0 + kk * DR, 8)
        pltpu.async_copy(stages[b], out.at[c, pl.ds(r, DR)], semw)

    def wait_rd(b):
        pltpu.make_async_copy(acc.at[pl.ds(0, DR)], stages[b], semr).wait()

    def wait_wr(b):
        pltpu.make_async_copy(stages[b], out.at[c, pl.ds(0, DR)], semw).wait()

    rd(0, 0)
    rd(1, 1)

    def pair(pp, _):
        k0 = pp * 2
        for j in range(2):
            kk = k0 + j
            wait_rd(j)
            wr(kk, j)

            @pl.when(kk + 2 < NDR)
            def _():
                wait_wr(j)
                rd(kk + 2, j)
        return 0
    lax.fori_loop(0, NDR // 2, pair, 0)
    wait_wr(0)
    wait_wr(1)


def _make_seg_sum():
    @functools.partial(
        pl.kernel,
        out_type=jax.ShapeDtypeStruct((NC, N_PAD, FD), jnp.float32),
        mesh=_mk_mesh(),
        scratch_types=[
            [pltpu.VMEM((CHUNK,), jnp.int32) for _ in range(4)],
            [pltpu.VMEM((CHUNK,), jnp.int32) for _ in range(4)],
            [pltpu.VMEM((CHUNK, FD), jnp.float32) for _ in range(2)],
            [pltpu.VMEM((DR, FD), jnp.float32) for _ in range(2)],
            pltpu.VMEM_SHARED((N_PAD, FD), jnp.float32),
            pltpu.SemaphoreType.DMA,
            pltpu.SemaphoreType.DMA,
            pltpu.SemaphoreType.DMA,
            pltpu.SemaphoreType.DMA,
        ],
    )
    def k(table, src3, dst3, out, srcs, dsts, rows, stages, acc, semg, semi,
          semr, semw):
        c = lax.axis_index("c")
        s = lax.axis_index("s")
        wid = c * NS + s
        zero16 = jnp.zeros((16,), jnp.float32)

        def zrow(i, _):
            r = i // (FD // 16)
            q = i % (FD // 16)
            stages[0][r, pl.ds(q * 16, 16)] = zero16
            return 0
        lax.fori_loop(0, DR * (FD // 16), zrow, 0)

        row0 = s * RPT

        def zcp(kk, _):
            r = pl.multiple_of(row0 + kk * DR, 8)
            pltpu.async_copy(stages[0], acc.at[pl.ds(r, DR)], semw)
            return 0
        lax.fori_loop(0, NDR, zcp, 0)

        def zwait(kk, _):
            pltpu.make_async_copy(acc.at[pl.ds(0, DR)], stages[0], semw).wait()
            return 0
        lax.fori_loop(0, NDR, zwait, 0)
        plsc.subcore_barrier()

        pltpu.sync_copy(src3.at[wid, 0], srcs[0])
        pltpu.sync_copy(dst3.at[wid, 0], dsts[0])
        pltpu.sync_copy(src3.at[wid, 1], srcs[1])
        pltpu.sync_copy(dst3.at[wid, 1], dsts[1])
        pltpu.async_copy(table.at[srcs[0]], rows[0], semg)
        pltpu.async_copy(table.at[srcs[1]], rows[1], semg)
        pltpu.async_copy(src3.at[wid, 2], srcs[2], semi)
        pltpu.async_copy(dst3.at[wid, 2], dsts[2], semi)
        pltpu.async_copy(src3.at[wid, 3], srcs[3], semi)
        pltpu.async_copy(dst3.at[wid, 3], dsts[3], semi)

        def quad(qq, _):
            q = qq * 4
            for j in range(4):
                ch = q + j
                rb = rows[j % 2]
                ib = j

                @pl.when(ch < NCHUNK)
                def _():
                    pltpu.make_async_copy(table.at[pl.ds(0, CHUNK)], rb, semg).wait()
                    pltpu.sync_copy(rb, acc.at[dsts[ib]], add=True)

                @pl.when(ch + 4 < NCHUNK)
                def _():
                    pltpu.async_copy(src3.at[wid, ch + 4], srcs[ib], semi)
                    pltpu.async_copy(dst3.at[wid, ch + 4], dsts[ib], semi)

                @pl.when(ch + 2 < NCHUNK)
                def _():
                    ib2 = (j + 2) % 4
                    pltpu.make_async_copy(src3.at[wid, 0], srcs[ib2], semi).wait()
                    pltpu.make_async_copy(dst3.at[wid, 0], dsts[ib2], semi).wait()
                    pltpu.async_copy(table.at[srcs[ib2]], rb, semg)
            return 0
        lax.fori_loop(0, NQ, quad, 0)
        plsc.subcore_barrier()
        _drain_acc(c, row0, acc, out, stages, semr, semw)

    return k


def _make_count():
    @functools.partial(
        pl.kernel,
        out_type=jax.ShapeDtypeStruct((NC, N_PAD, FD), jnp.float32),
        mesh=_mk_mesh(),
        scratch_types=[
            [pltpu.VMEM((CHUNK,), jnp.int32) for _ in range(4)],
            pltpu.VMEM((CHUNK, FD), jnp.float32),
            [pltpu.VMEM((DR, FD), jnp.float32) for _ in range(2)],
            pltpu.VMEM_SHARED((N_PAD, FD), jnp.float32),
            pltpu.SemaphoreType.DMA,
            pltpu.SemaphoreType.DMA,
            pltpu.SemaphoreType.DMA,
        ],
    )
    def k(dst3, out, dsts, ones_v, stages, acc, semi, semr, semw):
        c = lax.axis_index("c")
        s = lax.axis_index("s")
        wid = c * NS + s
        zero16 = jnp.zeros((16,), jnp.float32)
        one16 = jnp.ones((16,), jnp.float32)

        def zrow(i, _):
            r = i // (FD // 16)
            q = i % (FD // 16)
            stages[0][r, pl.ds(q * 16, 16)] = zero16
            return 0
        lax.fori_loop(0, DR * (FD // 16), zrow, 0)

        def orow(i, _):
            r = i // (FD // 16)
            q = i % (FD // 16)
            ones_v[r, pl.ds(q * 16, 16)] = one16
            return 0
        lax.fori_loop(0, CHUNK * (FD // 16), orow, 0)

        row0 = s * RPT

        def zcp(kk, _):
            r = pl.multiple_of(row0 + kk * DR, 8)
            pltpu.async_copy(stages[0], acc.at[pl.ds(r, DR)], semw)
            return 0
        lax.fori_loop(0, NDR, zcp, 0)

        def zwait(kk, _):
            pltpu.make_async_copy(acc.at[pl.ds(0, DR)], stages[0], semw).wait()
            return 0
        lax.fori_loop(0, NDR, zwait, 0)
        plsc.subcore_barrier()

        for j in range(4):
            pltpu.async_copy(dst3.at[wid, j], dsts[j], semi)

        def quad(qq, _):
            q = qq * 4
            for j in range(4):
                ch = q + j

                @pl.when(ch < NCHUNK)
                def _():
                    pltpu.make_async_copy(dst3.at[wid, 0], dsts[j], semi).wait()
                    pltpu.sync_copy(ones_v, acc.at[dsts[j]], add=True)

                @pl.when(ch + 4 < NCHUNK)
                def _():
                    pltpu.async_copy(dst3.at[wid, ch + 4], dsts[j], semi)
            return 0
        lax.fori_loop(0, NQ, quad, 0)
        plsc.subcore_barrier()
        _drain_acc(c, row0, acc, out, stages, semr, semw)

    return k


_seg_sum = _make_seg_sum()
_count_seg = _make_count()


HALF = N_NODES // 2


def _project(x_src, x_dst, W_src, b_src, W_dst, b_dst):
    def body(xs, xd, ws, bs, wd, bd, o):
        o[0:HALF, :] = (
            jnp.dot(xs[...], ws[...], preferred_element_type=jnp.float32) + bs[...]
        )
        o[HALF:N_NODES, :] = (
            jnp.dot(xd[...], wd[...], preferred_element_type=jnp.float32) + bd[...]
        )

    return pl.pallas_call(
        body,
        out_shape=jax.ShapeDtypeStruct((N_NODES, FD), jnp.float32),
    )(x_src, x_dst, W_src, b_src.reshape(1, FD), W_dst, b_dst.reshape(1, FD))


RB = 400


def _layer1(parts, cnts, x, W1_l, b1_l, W1_r):
    grid = (N_NODES // RB,)

    def body(p_ref, c_ref, x_ref, wl, bl, wr, hpre_ref, stat_ref):
        i = pl.program_id(0)
        psum = p_ref[0] + p_ref[1]
        cnt = c_ref[0, :, 0:1] + c_ref[1, :, 0:1]
        inv = 1.0 / jnp.maximum(cnt, 1.0)
        agg = psum * inv
        hp = (
            jnp.dot(agg, wl[...], preferred_element_type=jnp.float32)
            + bl[...]
            + jnp.dot(x_ref[...], wr[...], preferred_element_type=jnp.float32)
        )
        hpre_ref[...] = hp

        @pl.when(i == 0)
        def _():
            stat_ref[...] = jnp.zeros_like(stat_ref)

        stat_ref[0:1, :] += jnp.sum(hp, axis=0, keepdims=True)
        stat_ref[1:2, :] += jnp.sum(hp * hp, axis=0, keepdims=True)

    return pl.pallas_call(
        body,
        grid=grid,
        in_specs=[
            pl.BlockSpec((2, RB, FD), lambda i: (0, i, 0)),
            pl.BlockSpec((2, RB, FD), lambda i: (0, i, 0)),
            pl.BlockSpec((RB, FD), lambda i: (i, 0)),
            pl.BlockSpec((FD, FD), lambda i: (0, 0)),
            pl.BlockSpec((1, FD), lambda i: (0, 0)),
            pl.BlockSpec((FD, FD), lambda i: (0, 0)),
        ],
        out_specs=[
            pl.BlockSpec((RB, FD), lambda i: (i, 0)),
            pl.BlockSpec((8, FD), lambda i: (0, 0)),
        ],
        out_shape=[
            jax.ShapeDtypeStruct((N_NODES, FD), jnp.float32),
            jax.ShapeDtypeStruct((8, FD), jnp.float32),
        ],
    )(parts, cnts, x, W1_l, b1_l.reshape(1, FD), W1_r)


def _bn_relu(hpre, stats, gamma, beta):
    grid = (N_NODES // RB,)

    def body(h_ref, st_ref, g_ref, b_ref, o_ref):
        mu = st_ref[0:1, :] / N_NODES
        var = st_ref[1:2, :] / N_NODES - mu * mu
        scale = g_ref[...] * lax.rsqrt(var + 1e-5)
        shift = b_ref[...] - mu * scale
        o_ref[...] = jnp.maximum(h_ref[...] * scale + shift, 0.0)

    return pl.pallas_call(
        body,
        grid=grid,
        in_specs=[
            pl.BlockSpec((RB, FD), lambda i: (i, 0)),
            pl.BlockSpec((8, FD), lambda i: (0, 0)),
            pl.BlockSpec((1, FD), lambda i: (0, 0)),
            pl.BlockSpec((1, FD), lambda i: (0, 0)),
        ],
        out_specs=pl.BlockSpec((RB, FD), lambda i: (i, 0)),
        out_shape=jax.ShapeDtypeStruct((N_NODES, FD), jnp.float32),
    )(hpre, stats, gamma.reshape(1, FD), beta.reshape(1, FD))


def _layer2(parts, cnts, h, W2_l, b2_l, W2_r):
    grid = (N_NODES // RB,)

    def body(p_ref, c_ref, h_ref, wl, bl, wr, o_ref):
        psum = p_ref[0] + p_ref[1]
        cnt = c_ref[0, :, 0:1] + c_ref[1, :, 0:1]
        inv = 1.0 / jnp.maximum(cnt, 1.0)
        agg = psum * inv
        o_ref[...] = (
            jnp.dot(agg, wl[...], preferred_element_type=jnp.float32)
            + bl[...]
            + jnp.dot(h_ref[...], wr[...], preferred_element_type=jnp.float32)
        )

    return pl.pallas_call(
        body,
        grid=grid,
        in_specs=[
            pl.BlockSpec((2, RB, FD), lambda i: (0, i, 0)),
            pl.BlockSpec((2, RB, FD), lambda i: (0, i, 0)),
            pl.BlockSpec((RB, FD), lambda i: (i, 0)),
            pl.BlockSpec((FD, FD), lambda i: (0, 0)),
            pl.BlockSpec((1, FD), lambda i: (0, 0)),
            pl.BlockSpec((FD, FD), lambda i: (0, 0)),
        ],
        out_specs=pl.BlockSpec((RB, FD), lambda i: (i, 0)),
        out_shape=jax.ShapeDtypeStruct((N_NODES, FD), jnp.float32),
    )(parts, cnts, h, W2_l, b2_l.reshape(1, FD), W2_r)


def kernel(x_src, x_dst, edge_index, W_src, b_src, W_dst, b_dst,
           W1_l, b1_l, W1_r, W2_l, b2_l, W2_r, gamma, beta):
    src = edge_index[0].reshape(NW, NCHUNK, CHUNK)
    dst = edge_index[1].reshape(NW, NCHUNK, CHUNK)
    x = _project(x_src, x_dst, W_src, b_src, W_dst, b_dst)
    cnts = _count_seg(dst)
    parts1 = _seg_sum(x, src, dst)
    hpre, stats = _layer1(parts1, cnts, x, W1_l, b1_l, W1_r)
    h = _bn_relu(hpre, stats, gamma, beta)
    parts2 = _seg_sum(h, src, dst)
    out = _layer2(parts2, cnts, h, W2_l, b2_l, W2_r)
    return out


# speedup vs baseline: 8.7938x; 1.0627x over previous
import functools

import jax
import jax.numpy as jnp
from jax import lax
from jax.experimental import pallas as pl
from jax.experimental.pallas import tpu as pltpu
from jax.experimental.pallas import tpu_sc as plsc

N_NODES = 10000
N_EDGES = 320000
FD = 128

NC, NS = 2, 16
NW = NC * NS
EPW = N_EDGES // NW
CHUNK = 80
NCHUNK = EPW // CHUNK
N_PAD = 10240
RPT = N_PAD // NS
DR = 64
NDR = RPT // DR
NQ = (NCHUNK + 3) // 4


def _mk_mesh():
    return plsc.VectorSubcoreMesh(
        core_axis_name="c", subcore_axis_name="s", num_cores=NC, num_subcores=NS
    )


def _drain_acc(c, row0, acc, out, stages, semr, semw):
    def rd(kk, b):
        r = pl.multiple_of(row0 + kk * DR, 8)
        pltpu.async_copy(acc.at[pl.ds(r, DR)], stages[b], semr)

    def wr(kk, b):
        r = pl.multiple_of(row0 + kk * DR, 8)
        pltpu.async_copy(stages[b], out.at[c, pl.ds(r, DR)], semw)

    def wait_rd(b):
        pltpu.make_async_copy(acc.at[pl.ds(0, DR)], stages[b], semr).wait()

    def wait_wr(b):
        pltpu.make_async_copy(stages[b], out.at[c, pl.ds(0, DR)], semw).wait()

    rd(0, 0)
    rd(1, 1)

    def pair(pp, _):
        k0 = pp * 2
        for j in range(2):
            kk = k0 + j
            wait_rd(j)
            wr(kk, j)

            @pl.when(kk + 2 < NDR)
            def _():
                wait_wr(j)
                rd(kk + 2, j)
        return 0
    lax.fori_loop(0, NDR // 2, pair, 0)
    wait_wr(0)
    wait_wr(1)


def _make_seg_sum():
    @functools.partial(
        pl.kernel,
        out_type=jax.ShapeDtypeStruct((NC, N_PAD, FD), jnp.float32),
        mesh=_mk_mesh(),
        scratch_types=[
            [pltpu.VMEM((CHUNK,), jnp.int32) for _ in range(4)],
            [pltpu.VMEM((CHUNK,), jnp.int32) for _ in range(4)],
            [pltpu.VMEM((CHUNK, FD), jnp.float32) for _ in range(2)],
            [pltpu.VMEM((DR, FD), jnp.float32) for _ in range(2)],
            pltpu.VMEM_SHARED((N_PAD, FD), jnp.float32),
            pltpu.SemaphoreType.DMA,
            pltpu.SemaphoreType.DMA,
            pltpu.SemaphoreType.DMA,
            pltpu.SemaphoreType.DMA,
        ],
    )
    def k(table, src3, dst3, out, srcs, dsts, rows, stages, acc, semg, semi,
          semr, semw):
        c = lax.axis_index("c")
        s = lax.axis_index("s")
        wid = c * NS + s
        zero16 = jnp.zeros((16,), jnp.float32)

        def zrow(i, _):
            r = i // (FD // 16)
            q = i % (FD // 16)
            stages[0][r, pl.ds(q * 16, 16)] = zero16
            return 0
        lax.fori_loop(0, DR * (FD // 16), zrow, 0)

        row0 = s * RPT

        def zcp(kk, _):
            r = pl.multiple_of(row0 + kk * DR, 8)
            pltpu.async_copy(stages[0], acc.at[pl.ds(r, DR)], semw)
            return 0
        lax.fori_loop(0, NDR, zcp, 0)

        def zwait(kk, _):
            pltpu.make_async_copy(acc.at[pl.ds(0, DR)], stages[0], semw).wait()
            return 0
        lax.fori_loop(0, NDR, zwait, 0)
        plsc.subcore_barrier()

        pltpu.sync_copy(src3.at[wid, 0], srcs[0])
        pltpu.sync_copy(dst3.at[wid, 0], dsts[0])
        pltpu.sync_copy(src3.at[wid, 1], srcs[1])
        pltpu.sync_copy(dst3.at[wid, 1], dsts[1])
        pltpu.async_copy(table.at[srcs[0]], rows[0], semg)
        pltpu.async_copy(table.at[srcs[1]], rows[1], semg)
        pltpu.async_copy(src3.at[wid, 2], srcs[2], semi)
        pltpu.async_copy(dst3.at[wid, 2], dsts[2], semi)
        pltpu.async_copy(src3.at[wid, 3], srcs[3], semi)
        pltpu.async_copy(dst3.at[wid, 3], dsts[3], semi)

        def quad(qq, _):
            q = qq * 4
            for j in range(4):
                ch = q + j
                rb = rows[j % 2]
                ib = j

                @pl.when(ch < NCHUNK)
                def _():
                    pltpu.make_async_copy(table.at[pl.ds(0, CHUNK)], rb, semg).wait()
                    pltpu.sync_copy(rb, acc.at[dsts[ib]], add=True)

                @pl.when(ch + 4 < NCHUNK)
                def _():
                    pltpu.async_copy(src3.at[wid, ch + 4], srcs[ib], semi)
                    pltpu.async_copy(dst3.at[wid, ch + 4], dsts[ib], semi)

                @pl.when(ch + 2 < NCHUNK)
                def _():
                    ib2 = (j + 2) % 4
                    pltpu.make_async_copy(src3.at[wid, 0], srcs[ib2], semi).wait()
                    pltpu.make_async_copy(dst3.at[wid, 0], dsts[ib2], semi).wait()
                    pltpu.async_copy(table.at[srcs[ib2]], rb, semg)
            return 0
        lax.fori_loop(0, NQ, quad, 0)
        plsc.subcore_barrier()
        _drain_acc(c, row0, acc, out, stages, semr, semw)

    return k


def _make_count():
    @functools.partial(
        pl.kernel,
        out_type=jax.ShapeDtypeStruct((NC, N_PAD, FD), jnp.float32),
        mesh=_mk_mesh(),
        scratch_types=[
            [pltpu.VMEM((CHUNK,), jnp.int32) for _ in range(4)],
            pltpu.VMEM((CHUNK, FD), jnp.float32),
            [pltpu.VMEM((DR, FD), jnp.float32) for _ in range(2)],
            pltpu.VMEM_SHARED((N_PAD, FD), jnp.float32),
            pltpu.SemaphoreType.DMA,
            pltpu.SemaphoreType.DMA,
            pltpu.SemaphoreType.DMA,
        ],
    )
    def k(dst3, out, dsts, ones_v, stages, acc, semi, semr, semw):
        c = lax.axis_index("c")
        s = lax.axis_index("s")
        wid = c * NS + s
        zero16 = jnp.zeros((16,), jnp.float32)
        one16 = jnp.ones((16,), jnp.float32)

        def zrow(i, _):
            r = i // (FD // 16)
            q = i % (FD // 16)
            stages[0][r, pl.ds(q * 16, 16)] = zero16
            return 0
        lax.fori_loop(0, DR * (FD // 16), zrow, 0)

        def orow(i, _):
            r = i // (FD // 16)
            q = i % (FD // 16)
            ones_v[r, pl.ds(q * 16, 16)] = one16
            return 0
        lax.fori_loop(0, CHUNK * (FD // 16), orow, 0)

        row0 = s * RPT

        def zcp(kk, _):
            r = pl.multiple_of(row0 + kk * DR, 8)
            pltpu.async_copy(stages[0], acc.at[pl.ds(r, DR)], semw)
            return 0
        lax.fori_loop(0, NDR, zcp, 0)

        def zwait(kk, _):
            pltpu.make_async_copy(acc.at[pl.ds(0, DR)], stages[0], semw).wait()
            return 0
        lax.fori_loop(0, NDR, zwait, 0)
        plsc.subcore_barrier()

        for j in range(4):
            pltpu.async_copy(dst3.at[wid, j], dsts[j], semi)

        def quad(qq, _):
            q = qq * 4
            for j in range(4):
                ch = q + j

                @pl.when(ch < NCHUNK)
                def _():
                    pltpu.make_async_copy(dst3.at[wid, 0], dsts[j], semi).wait()
                    pltpu.sync_copy(ones_v, acc.at[dsts[j]], add=True)

                @pl.when(ch + 4 < NCHUNK)
                def _():
                    pltpu.async_copy(dst3.at[wid, ch + 4], dsts[j], semi)
            return 0
        lax.fori_loop(0, NQ, quad, 0)
        plsc.subcore_barrier()
        _drain_acc(c, row0, acc, out, stages, semr, semw)

    return k


_seg_sum = _make_seg_sum()
_count_seg = _make_count()


HALF = N_NODES // 2


def _project(x_src, x_dst, W_src, b_src, W_dst, b_dst):
    def body(xs, xd, ws, bs, wd, bd, o):
        o[0:HALF, :] = (
            jnp.dot(xs[...], ws[...], preferred_element_type=jnp.float32) + bs[...]
        )
        o[HALF:N_NODES, :] = (
            jnp.dot(xd[...], wd[...], preferred_element_type=jnp.float32) + bd[...]
        )

    return pl.pallas_call(
        body,
        out_shape=jax.ShapeDtypeStruct((N_NODES, FD), jnp.float32),
    )(x_src, x_dst, W_src, b_src.reshape(1, FD), W_dst, b_dst.reshape(1, FD))


RB = 1000


def _layer1(parts, cnts, x, W1_l, b1_l, W1_r):
    grid = (N_NODES // RB,)

    def body(p_ref, c_ref, x_ref, wl, bl, wr, hpre_ref, stat_ref):
        i = pl.program_id(0)
        psum = p_ref[0] + p_ref[1]
        cnt = c_ref[0, :, 0:1] + c_ref[1, :, 0:1]
        inv = 1.0 / jnp.maximum(cnt, 1.0)
        agg = psum * inv
        hp = (
            jnp.dot(agg, wl[...], preferred_element_type=jnp.float32)
            + bl[...]
            + jnp.dot(x_ref[...], wr[...], preferred_element_type=jnp.float32)
        )
        hpre_ref[...] = hp

        @pl.when(i == 0)
        def _():
            stat_ref[...] = jnp.zeros_like(stat_ref)

        stat_ref[0:1, :] += jnp.sum(hp, axis=0, keepdims=True)
        stat_ref[1:2, :] += jnp.sum(hp * hp, axis=0, keepdims=True)

    return pl.pallas_call(
        body,
        grid=grid,
        in_specs=[
            pl.BlockSpec((2, RB, FD), lambda i: (0, i, 0)),
            pl.BlockSpec((2, RB, FD), lambda i: (0, i, 0)),
            pl.BlockSpec((RB, FD), lambda i: (i, 0)),
            pl.BlockSpec((FD, FD), lambda i: (0, 0)),
            pl.BlockSpec((1, FD), lambda i: (0, 0)),
            pl.BlockSpec((FD, FD), lambda i: (0, 0)),
        ],
        out_specs=[
            pl.BlockSpec((RB, FD), lambda i: (i, 0)),
            pl.BlockSpec((8, FD), lambda i: (0, 0)),
        ],
        out_shape=[
            jax.ShapeDtypeStruct((N_NODES, FD), jnp.float32),
            jax.ShapeDtypeStruct((8, FD), jnp.float32),
        ],
    )(parts, cnts, x, W1_l, b1_l.reshape(1, FD), W1_r)


def _bn_relu(hpre, stats, gamma, beta):
    grid = (N_NODES // RB,)

    def body(h_ref, st_ref, g_ref, b_ref, o_ref):
        mu = st_ref[0:1, :] / N_NODES
        var = st_ref[1:2, :] / N_NODES - mu * mu
        scale = g_ref[...] * lax.rsqrt(var + 1e-5)
        shift = b_ref[...] - mu * scale
        o_ref[...] = jnp.maximum(h_ref[...] * scale + shift, 0.0)

    return pl.pallas_call(
        body,
        grid=grid,
        in_specs=[
            pl.BlockSpec((RB, FD), lambda i: (i, 0)),
            pl.BlockSpec((8, FD), lambda i: (0, 0)),
            pl.BlockSpec((1, FD), lambda i: (0, 0)),
            pl.BlockSpec((1, FD), lambda i: (0, 0)),
        ],
        out_specs=pl.BlockSpec((RB, FD), lambda i: (i, 0)),
        out_shape=jax.ShapeDtypeStruct((N_NODES, FD), jnp.float32),
    )(hpre, stats, gamma.reshape(1, FD), beta.reshape(1, FD))


def _layer2(parts, cnts, h, W2_l, b2_l, W2_r):
    grid = (N_NODES // RB,)

    def body(p_ref, c_ref, h_ref, wl, bl, wr, o_ref):
        psum = p_ref[0] + p_ref[1]
        cnt = c_ref[0, :, 0:1] + c_ref[1, :, 0:1]
        inv = 1.0 / jnp.maximum(cnt, 1.0)
        agg = psum * inv
        o_ref[...] = (
            jnp.dot(agg, wl[...], preferred_element_type=jnp.float32)
            + bl[...]
            + jnp.dot(h_ref[...], wr[...], preferred_element_type=jnp.float32)
        )

    return pl.pallas_call(
        body,
        grid=grid,
        in_specs=[
            pl.BlockSpec((2, RB, FD), lambda i: (0, i, 0)),
            pl.BlockSpec((2, RB, FD), lambda i: (0, i, 0)),
            pl.BlockSpec((RB, FD), lambda i: (i, 0)),
            pl.BlockSpec((FD, FD), lambda i: (0, 0)),
            pl.BlockSpec((1, FD), lambda i: (0, 0)),
            pl.BlockSpec((FD, FD), lambda i: (0, 0)),
        ],
        out_specs=pl.BlockSpec((RB, FD), lambda i: (i, 0)),
        out_shape=jax.ShapeDtypeStruct((N_NODES, FD), jnp.float32),
    )(parts, cnts, h, W2_l, b2_l.reshape(1, FD), W2_r)


def kernel(x_src, x_dst, edge_index, W_src, b_src, W_dst, b_dst,
           W1_l, b1_l, W1_r, W2_l, b2_l, W2_r, gamma, beta):
    src = edge_index[0].reshape(NW, NCHUNK, CHUNK)
    dst = edge_index[1].reshape(NW, NCHUNK, CHUNK)
    x = _project(x_src, x_dst, W_src, b_src, W_dst, b_dst)
    cnts = _count_seg(dst)
    parts1 = _seg_sum(x, src, dst)
    hpre, stats = _layer1(parts1, cnts, x, W1_l, b1_l, W1_r)
    h = _bn_relu(hpre, stats, gamma, beta)
    parts2 = _seg_sum(h, src, dst)
    out = _layer2(parts2, cnts, h, W2_l, b2_l, W2_r)
    return out


# speedup vs baseline: 8.9632x; 1.0193x over previous
import functools

import jax
import jax.numpy as jnp
from jax import lax
from jax.experimental import pallas as pl
from jax.experimental.pallas import tpu as pltpu
from jax.experimental.pallas import tpu_sc as plsc

N_NODES = 10000
N_EDGES = 320000
FD = 128

NC, NS = 2, 16
NW = NC * NS
EPW = N_EDGES // NW
CHUNK = 80
NCHUNK = EPW // CHUNK
N_PAD = 10240
RPT = N_PAD // NS
DR = 32
NDR = RPT // DR
NQ = (NCHUNK + 3) // 4


def _mk_mesh():
    return plsc.VectorSubcoreMesh(
        core_axis_name="c", subcore_axis_name="s", num_cores=NC, num_subcores=NS
    )


def _drain_acc(c, row0, acc, out, stages, semr, semw):
    def rd(kk, b):
        r = pl.multiple_of(row0 + kk * DR, 8)
        pltpu.async_copy(acc.at[pl.ds(r, DR)], stages[b], semr)

    def wr(kk, b):
        r = pl.multiple_of(row0 + kk * DR, 8)
        pltpu.async_copy(stages[b], out.at[c, pl.ds(r, DR)], semw)

    def wait_rd(b):
        pltpu.make_async_copy(acc.at[pl.ds(0, DR)], stages[b], semr).wait()

    def wait_wr(b):
        pltpu.make_async_copy(stages[b], out.at[c, pl.ds(0, DR)], semw).wait()

    rd(0, 0)
    rd(1, 1)

    def pair(pp, _):
        k0 = pp * 2
        for j in range(2):
            kk = k0 + j
            wait_rd(j)
            wr(kk, j)

            @pl.when(kk + 2 < NDR)
            def _():
                wait_wr(j)
                rd(kk + 2, j)
        return 0
    lax.fori_loop(0, NDR // 2, pair, 0)
    wait_wr(0)
    wait_wr(1)


def _make_seg_sum():
    @functools.partial(
        pl.kernel,
        out_type=jax.ShapeDtypeStruct((NC, N_PAD, FD), jnp.float32),
        mesh=_mk_mesh(),
        scratch_types=[
            [pltpu.VMEM((CHUNK,), jnp.int32) for _ in range(4)],
            [pltpu.VMEM((CHUNK,), jnp.int32) for _ in range(4)],
            [pltpu.VMEM((CHUNK, FD), jnp.float32) for _ in range(2)],
            [pltpu.VMEM((DR, FD), jnp.float32) for _ in range(2)],
            pltpu.VMEM_SHARED((N_PAD, FD), jnp.float32),
            pltpu.SemaphoreType.DMA,
            pltpu.SemaphoreType.DMA,
            pltpu.SemaphoreType.DMA,
            pltpu.SemaphoreType.DMA,
        ],
    )
    def k(table, src3, dst3, out, srcs, dsts, rows, stages, acc, semg, semi,
          semr, semw):
        c = lax.axis_index("c")
        s = lax.axis_index("s")
        wid = c * NS + s
        zero16 = jnp.zeros((16,), jnp.float32)

        def zrow(i, _):
            r = i // (FD // 16)
            q = i % (FD // 16)
            stages[0][r, pl.ds(q * 16, 16)] = zero16
            return 0
        lax.fori_loop(0, DR * (FD // 16), zrow, 0)

        row0 = s * RPT

        def zcp(kk, _):
            r = pl.multiple_of(row0 + kk * DR, 8)
            pltpu.async_copy(stages[0], acc.at[pl.ds(r, DR)], semw)
            return 0
        lax.fori_loop(0, NDR, zcp, 0)

        def zwait(kk, _):
            pltpu.make_async_copy(acc.at[pl.ds(0, DR)], stages[0], semw).wait()
            return 0
        lax.fori_loop(0, NDR, zwait, 0)
        plsc.subcore_barrier()

        pltpu.sync_copy(src3.at[wid, 0], srcs[0])
        pltpu.sync_copy(dst3.at[wid, 0], dsts[0])
        pltpu.sync_copy(src3.at[wid, 1], srcs[1])
        pltpu.sync_copy(dst3.at[wid, 1], dsts[1])
        pltpu.async_copy(table.at[srcs[0]], rows[0], semg)
        pltpu.async_copy(table.at[srcs[1]], rows[1], semg)
        pltpu.async_copy(src3.at[wid, 2], srcs[2], semi)
        pltpu.async_copy(dst3.at[wid, 2], dsts[2], semi)
        pltpu.async_copy(src3.at[wid, 3], srcs[3], semi)
        pltpu.async_copy(dst3.at[wid, 3], dsts[3], semi)

        def quad(qq, _):
            q = qq * 4
            for j in range(4):
                ch = q + j
                rb = rows[j % 2]
                ib = j

                @pl.when(ch < NCHUNK)
                def _():
                    pltpu.make_async_copy(table.at[pl.ds(0, CHUNK)], rb, semg).wait()
                    pltpu.sync_copy(rb, acc.at[dsts[ib]], add=True)

                @pl.when(ch + 4 < NCHUNK)
                def _():
                    pltpu.async_copy(src3.at[wid, ch + 4], srcs[ib], semi)
                    pltpu.async_copy(dst3.at[wid, ch + 4], dsts[ib], semi)

                @pl.when(ch + 2 < NCHUNK)
                def _():
                    ib2 = (j + 2) % 4
                    pltpu.make_async_copy(src3.at[wid, 0], srcs[ib2], semi).wait()
                    pltpu.make_async_copy(dst3.at[wid, 0], dsts[ib2], semi).wait()
                    pltpu.async_copy(table.at[srcs[ib2]], rb, semg)
            return 0
        lax.fori_loop(0, NQ, quad, 0)
        plsc.subcore_barrier()
        _drain_acc(c, row0, acc, out, stages, semr, semw)

    return k


def _make_count_seg():
    @functools.partial(
        pl.kernel,
        out_type=[jax.ShapeDtypeStruct((NC, N_PAD, FD), jnp.float32),
                  jax.ShapeDtypeStruct((NC, N_PAD, FD), jnp.float32)],
        mesh=_mk_mesh(),
        scratch_types=[
            [pltpu.VMEM((CHUNK,), jnp.int32) for _ in range(4)],
            [pltpu.VMEM((CHUNK,), jnp.int32) for _ in range(4)],
            [pltpu.VMEM((CHUNK, FD), jnp.float32) for _ in range(2)],
            [pltpu.VMEM((DR, FD), jnp.float32) for _ in range(2)],
            pltpu.VMEM_SHARED((N_PAD, FD), jnp.float32),
            pltpu.SemaphoreType.DMA,
            pltpu.SemaphoreType.DMA,
            pltpu.SemaphoreType.DMA,
            pltpu.SemaphoreType.DMA,
        ],
    )
    def k(table, src3, dst3, out, cnt_out, srcs, dsts, rows, stages, acc,
          semg, semi, semr, semw):
        c = lax.axis_index("c")
        s = lax.axis_index("s")
        wid = c * NS + s
        zero16 = jnp.zeros((16,), jnp.float32)
        one16 = jnp.ones((16,), jnp.float32)

        def zrow(i, _):
            r = i // (FD // 16)
            q = i % (FD // 16)
            stages[0][r, pl.ds(q * 16, 16)] = zero16
            return 0
        lax.fori_loop(0, DR * (FD // 16), zrow, 0)

        def orow(i, _):
            r = i // (FD // 16)
            q = i % (FD // 16)
            rows[0][r, pl.ds(q * 16, 16)] = one16
            return 0
        lax.fori_loop(0, CHUNK * (FD // 16), orow, 0)

        row0 = s * RPT

        def zcp(kk, _):
            r = pl.multiple_of(row0 + kk * DR, 8)
            pltpu.async_copy(stages[0], acc.at[pl.ds(r, DR)], semw)
            return 0
        lax.fori_loop(0, NDR, zcp, 0)

        def zwait(kk, _):
            pltpu.make_async_copy(acc.at[pl.ds(0, DR)], stages[0], semw).wait()
            return 0
        lax.fori_loop(0, NDR, zwait, 0)
        plsc.subcore_barrier()

        for j in range(4):
            pltpu.async_copy(dst3.at[wid, j], dsts[j], semi)

        def cquad(qq, _):
            q = qq * 4
            for j in range(4):
                ch = q + j

                @pl.when(ch < NCHUNK)
                def _():
                    pltpu.make_async_copy(dst3.at[wid, 0], dsts[j], semi).wait()
                    pltpu.sync_copy(rows[0], acc.at[dsts[j]], add=True)

                @pl.when(ch + 4 < NCHUNK)
                def _():
                    pltpu.async_copy(dst3.at[wid, ch + 4], dsts[j], semi)
            return 0
        lax.fori_loop(0, NQ, cquad, 0)
        plsc.subcore_barrier()

        pltpu.sync_copy(src3.at[wid, 0], srcs[0])
        pltpu.sync_copy(dst3.at[wid, 0], dsts[0])
        pltpu.sync_copy(src3.at[wid, 1], srcs[1])
        pltpu.sync_copy(dst3.at[wid, 1], dsts[1])
        pltpu.async_copy(table.at[srcs[0]], rows[0], semg)
        pltpu.async_copy(table.at[srcs[1]], rows[1], semg)
        pltpu.async_copy(src3.at[wid, 2], srcs[2], semi)
        pltpu.async_copy(dst3.at[wid, 2], dsts[2], semi)
        pltpu.async_copy(src3.at[wid, 3], srcs[3], semi)
        pltpu.async_copy(dst3.at[wid, 3], dsts[3], semi)

        _drain_acc(c, row0, acc, cnt_out, stages, semr, semw)
        lax.fori_loop(0, DR * (FD // 16), zrow, 0)
        lax.fori_loop(0, NDR, zcp, 0)
        lax.fori_loop(0, NDR, zwait, 0)
        plsc.subcore_barrier()

        def quad(qq, _):
            q = qq * 4
            for j in range(4):
                ch = q + j
                rb = rows[j % 2]
                ib = j

                @pl.when(ch < NCHUNK)
                def _():
                    pltpu.make_async_copy(table.at[pl.ds(0, CHUNK)], rb, semg).wait()
                    pltpu.sync_copy(rb, acc.at[dsts[ib]], add=True)

                @pl.when(ch + 4 < NCHUNK)
                def _():
                    pltpu.async_copy(src3.at[wid, ch + 4], srcs[ib], semi)
                    pltpu.async_copy(dst3.at[wid, ch + 4], dsts[ib], semi)

                @pl.when(ch + 2 < NCHUNK)
                def _():
                    ib2 = (j + 2) % 4
                    pltpu.make_async_copy(src3.at[wid, 0], srcs[ib2], semi).wait()
                    pltpu.make_async_copy(dst3.at[wid, 0], dsts[ib2], semi).wait()
                    pltpu.async_copy(table.at[srcs[ib2]], rb, semg)
            return 0
        lax.fori_loop(0, NQ, quad, 0)
        plsc.subcore_barrier()
        _drain_acc(c, row0, acc, out, stages, semr, semw)

    return k


def _make_count():
    @functools.partial(
        pl.kernel,
        out_type=jax.ShapeDtypeStruct((NC, N_PAD, FD), jnp.float32),
        mesh=_mk_mesh(),
        scratch_types=[
            [pltpu.VMEM((CHUNK,), jnp.int32) for _ in range(4)],
            pltpu.VMEM((CHUNK, FD), jnp.float32),
            [pltpu.VMEM((DR, FD), jnp.float32) for _ in range(2)],
            pltpu.VMEM_SHARED((N_PAD, FD), jnp.float32),
            pltpu.SemaphoreType.DMA,
            pltpu.SemaphoreType.DMA,
            pltpu.SemaphoreType.DMA,
        ],
    )
    def k(dst3, out, dsts, ones_v, stages, acc, semi, semr, semw):
        c = lax.axis_index("c")
        s = lax.axis_index("s")
        wid = c * NS + s
        zero16 = jnp.zeros((16,), jnp.float32)
        one16 = jnp.ones((16,), jnp.float32)

        def zrow(i, _):
            r = i // (FD // 16)
            q = i % (FD // 16)
            stages[0][r, pl.ds(q * 16, 16)] = zero16
            return 0
        lax.fori_loop(0, DR * (FD // 16), zrow, 0)

        def orow(i, _):
            r = i // (FD // 16)
            q = i % (FD // 16)
            ones_v[r, pl.ds(q * 16, 16)] = one16
            return 0
        lax.fori_loop(0, CHUNK * (FD // 16), orow, 0)

        row0 = s * RPT

        def zcp(kk, _):
            r = pl.multiple_of(row0 + kk * DR, 8)
            pltpu.async_copy(stages[0], acc.at[pl.ds(r, DR)], semw)
            return 0
        lax.fori_loop(0, NDR, zcp, 0)

        def zwait(kk, _):
            pltpu.make_async_copy(acc.at[pl.ds(0, DR)], stages[0], semw).wait()
            return 0
        lax.fori_loop(0, NDR, zwait, 0)
        plsc.subcore_barrier()

        for j in range(4):
            pltpu.async_copy(dst3.at[wid, j], dsts[j], semi)

        def quad(qq, _):
            q = qq * 4
            for j in range(4):
                ch = q + j

                @pl.when(ch < NCHUNK)
                def _():
                    pltpu.make_async_copy(dst3.at[wid, 0], dsts[j], semi).wait()
                    pltpu.sync_copy(ones_v, acc.at[dsts[j]], add=True)

                @pl.when(ch + 4 < NCHUNK)
                def _():
                    pltpu.async_copy(dst3.at[wid, ch + 4], dsts[j], semi)
            return 0
        lax.fori_loop(0, NQ, quad, 0)
        plsc.subcore_barrier()
        _drain_acc(c, row0, acc, out, stages, semr, semw)

    return k


_seg_sum = _make_seg_sum()
_count_seg_sum = _make_count_seg()


HALF = N_NODES // 2


def _project(x_src, x_dst, W_src, b_src, W_dst, b_dst):
    def body(xs, xd, ws, bs, wd, bd, o):
        o[0:HALF, :] = (
            jnp.dot(xs[...], ws[...], preferred_element_type=jnp.float32) + bs[...]
        )
        o[HALF:N_NODES, :] = (
            jnp.dot(xd[...], wd[...], preferred_element_type=jnp.float32) + bd[...]
        )

    return pl.pallas_call(
        body,
        out_shape=jax.ShapeDtypeStruct((N_NODES, FD), jnp.float32),
    )(x_src, x_dst, W_src, b_src.reshape(1, FD), W_dst, b_dst.reshape(1, FD))


RB = 1000


def _layer1(parts, cnts, x, W1_l, b1_l, W1_r):
    grid = (N_NODES // RB,)

    def body(p_ref, c_ref, x_ref, wl, bl, wr, hpre_ref, stat_ref):
        i = pl.program_id(0)
        psum = p_ref[0] + p_ref[1]
        cnt = c_ref[0, :, 0:1] + c_ref[1, :, 0:1]
        inv = 1.0 / jnp.maximum(cnt, 1.0)
        agg = psum * inv
        hp = (
            jnp.dot(agg, wl[...], preferred_element_type=jnp.float32)
            + bl[...]
            + jnp.dot(x_ref[...], wr[...], preferred_element_type=jnp.float32)
        )
        hpre_ref[...] = hp

        @pl.when(i == 0)
        def _():
            stat_ref[...] = jnp.zeros_like(stat_ref)

        stat_ref[0:1, :] += jnp.sum(hp, axis=0, keepdims=True)
        stat_ref[1:2, :] += jnp.sum(hp * hp, axis=0, keepdims=True)

    return pl.pallas_call(
        body,
        grid=grid,
        in_specs=[
            pl.BlockSpec((2, RB, FD), lambda i: (0, i, 0)),
            pl.BlockSpec((2, RB, FD), lambda i: (0, i, 0)),
            pl.BlockSpec((RB, FD), lambda i: (i, 0)),
            pl.BlockSpec((FD, FD), lambda i: (0, 0)),
            pl.BlockSpec((1, FD), lambda i: (0, 0)),
            pl.BlockSpec((FD, FD), lambda i: (0, 0)),
        ],
        out_specs=[
            pl.BlockSpec((RB, FD), lambda i: (i, 0)),
            pl.BlockSpec((8, FD), lambda i: (0, 0)),
        ],
        out_shape=[
            jax.ShapeDtypeStruct((N_NODES, FD), jnp.float32),
            jax.ShapeDtypeStruct((8, FD), jnp.float32),
        ],
    )(parts, cnts, x, W1_l, b1_l.reshape(1, FD), W1_r)


def _bn_relu(hpre, stats, gamma, beta):
    grid = (N_NODES // RB,)

    def body(h_ref, st_ref, g_ref, b_ref, o_ref):
        mu = st_ref[0:1, :] / N_NODES
        var = st_ref[1:2, :] / N_NODES - mu * mu
        scale = g_ref[...] * lax.rsqrt(var + 1e-5)
        shift = b_ref[...] - mu * scale
        o_ref[...] = jnp.maximum(h_ref[...] * scale + shift, 0.0)

    return pl.pallas_call(
        body,
        grid=grid,
        in_specs=[
            pl.BlockSpec((RB, FD), lambda i: (i, 0)),
            pl.BlockSpec((8, FD), lambda i: (0, 0)),
            pl.BlockSpec((1, FD), lambda i: (0, 0)),
            pl.BlockSpec((1, FD), lambda i: (0, 0)),
        ],
        out_specs=pl.BlockSpec((RB, FD), lambda i: (i, 0)),
        out_shape=jax.ShapeDtypeStruct((N_NODES, FD), jnp.float32),
    )(hpre, stats, gamma.reshape(1, FD), beta.reshape(1, FD))


def _layer2(parts, cnts, h, W2_l, b2_l, W2_r):
    grid = (N_NODES // RB,)

    def body(p_ref, c_ref, h_ref, wl, bl, wr, o_ref):
        psum = p_ref[0] + p_ref[1]
        cnt = c_ref[0, :, 0:1] + c_ref[1, :, 0:1]
        inv = 1.0 / jnp.maximum(cnt, 1.0)
        agg = psum * inv
        o_ref[...] = (
            jnp.dot(agg, wl[...], preferred_element_type=jnp.float32)
            + bl[...]
            + jnp.dot(h_ref[...], wr[...], preferred_element_type=jnp.float32)
        )

    return pl.pallas_call(
        body,
        grid=grid,
        in_specs=[
            pl.BlockSpec((2, RB, FD), lambda i: (0, i, 0)),
            pl.BlockSpec((2, RB, FD), lambda i: (0, i, 0)),
            pl.BlockSpec((RB, FD), lambda i: (i, 0)),
            pl.BlockSpec((FD, FD), lambda i: (0, 0)),
            pl.BlockSpec((1, FD), lambda i: (0, 0)),
            pl.BlockSpec((FD, FD), lambda i: (0, 0)),
        ],
        out_specs=pl.BlockSpec((RB, FD), lambda i: (i, 0)),
        out_shape=jax.ShapeDtypeStruct((N_NODES, FD), jnp.float32),
    )(parts, cnts, h, W2_l, b2_l.reshape(1, FD), W2_r)


def kernel(x_src, x_dst, edge_index, W_src, b_src, W_dst, b_dst,
           W1_l, b1_l, W1_r, W2_l, b2_l, W2_r, gamma, beta):
    src = edge_index[0].reshape(NW, NCHUNK, CHUNK)
    dst = edge_index[1].reshape(NW, NCHUNK, CHUNK)
    x = _project(x_src, x_dst, W_src, b_src, W_dst, b_dst)
    parts1, cnts = _count_seg_sum(x, src, dst)
    hpre, stats = _layer1(parts1, cnts, x, W1_l, b1_l, W1_r)
    h = _bn_relu(hpre, stats, gamma, beta)
    parts2 = _seg_sum(h, src, dst)
    out = _layer2(parts2, cnts, h, W2_l, b2_l, W2_r)
    return out


# speedup vs baseline: 9.2167x; 1.0283x over previous
import functools

import jax
import jax.numpy as jnp
from jax import lax
from jax.experimental import pallas as pl
from jax.experimental.pallas import tpu as pltpu
from jax.experimental.pallas import tpu_sc as plsc

N_NODES = 10000
N_EDGES = 320000
FD = 128

NC, NS = 2, 16
NW = NC * NS
EPW = N_EDGES // NW
CHUNK = 80
NCHUNK = EPW // CHUNK
N_PAD = 10240
RPT = N_PAD // NS
DR = 32
NDR = RPT // DR
NQ = (NCHUNK + 3) // 4


def _mk_mesh():
    return plsc.VectorSubcoreMesh(
        core_axis_name="c", subcore_axis_name="s", num_cores=NC, num_subcores=NS
    )


def _drain_acc(c, row0, acc, out, stages, semr, semw):
    def rd(kk, b):
        r = pl.multiple_of(row0 + kk * DR, 8)
        pltpu.async_copy(acc.at[pl.ds(r, DR)], stages[b], semr)

    def wr(kk, b):
        r = pl.multiple_of(row0 + kk * DR, 8)
        pltpu.async_copy(stages[b], out.at[c, pl.ds(r, DR)], semw)

    def wait_rd(b):
        pltpu.make_async_copy(acc.at[pl.ds(0, DR)], stages[b], semr).wait()

    def wait_wr(b):
        pltpu.make_async_copy(stages[b], out.at[c, pl.ds(0, DR)], semw).wait()

    rd(0, 0)
    rd(1, 1)

    def pair(pp, _):
        k0 = pp * 2
        for j in range(2):
            kk = k0 + j
            wait_rd(j)
            wr(kk, j)

            @pl.when(kk + 2 < NDR)
            def _():
                wait_wr(j)
                rd(kk + 2, j)
        return 0
    lax.fori_loop(0, NDR // 2, pair, 0)
    wait_wr(0)
    wait_wr(1)


def _make_seg_sum():
    @functools.partial(
        pl.kernel,
        out_type=jax.ShapeDtypeStruct((NC, N_PAD, FD), jnp.float32),
        mesh=_mk_mesh(),
        scratch_types=[
            [pltpu.VMEM((CHUNK,), jnp.int32) for _ in range(4)],
            [pltpu.VMEM((CHUNK,), jnp.int32) for _ in range(4)],
            [pltpu.VMEM((CHUNK, FD), jnp.float32) for _ in range(2)],
            [pltpu.VMEM((DR, FD), jnp.float32) for _ in range(2)],
            pltpu.VMEM_SHARED((N_PAD, FD), jnp.float32),
            pltpu.SemaphoreType.DMA,
            pltpu.SemaphoreType.DMA,
            pltpu.SemaphoreType.DMA,
            pltpu.SemaphoreType.DMA,
        ],
    )
    def k(table, src3, dst3, out, srcs, dsts, rows, stages, acc, semg, semi,
          semr, semw):
        c = lax.axis_index("c")
        s = lax.axis_index("s")
        wid = c * NS + s
        zero16 = jnp.zeros((16,), jnp.float32)

        def zrow(i, _):
            r = i // (FD // 16)
            q = i % (FD // 16)
            stages[0][r, pl.ds(q * 16, 16)] = zero16
            return 0
        lax.fori_loop(0, DR * (FD // 16), zrow, 0)

        row0 = s * RPT

        def zcp(kk, _):
            r = pl.multiple_of(row0 + kk * DR, 8)
            pltpu.async_copy(stages[0], acc.at[pl.ds(r, DR)], semw)
            return 0
        lax.fori_loop(0, NDR, zcp, 0)

        pltpu.sync_copy(src3.at[wid, 0], srcs[0])
        pltpu.sync_copy(dst3.at[wid, 0], dsts[0])
        pltpu.sync_copy(src3.at[wid, 1], srcs[1])
        pltpu.sync_copy(dst3.at[wid, 1], dsts[1])
        pltpu.async_copy(table.at[srcs[0]], rows[0], semg)
        pltpu.async_copy(table.at[srcs[1]], rows[1], semg)
        pltpu.async_copy(src3.at[wid, 2], srcs[2], semi)
        pltpu.async_copy(dst3.at[wid, 2], dsts[2], semi)
        pltpu.async_copy(src3.at[wid, 3], srcs[3], semi)
        pltpu.async_copy(dst3.at[wid, 3], dsts[3], semi)

        def zwait(kk, _):
            pltpu.make_async_copy(acc.at[pl.ds(0, DR)], stages[0], semw).wait()
            return 0
        lax.fori_loop(0, NDR, zwait, 0)
        plsc.subcore_barrier()

        def quad(qq, _):
            q = qq * 4
            for j in range(4):
                ch = q + j
                rb = rows[j % 2]
                ib = j

                @pl.when(ch < NCHUNK)
                def _():
                    pltpu.make_async_copy(table.at[pl.ds(0, CHUNK)], rb, semg).wait()
                    pltpu.sync_copy(rb, acc.at[dsts[ib]], add=True)

                @pl.when(ch + 4 < NCHUNK)
                def _():
                    pltpu.async_copy(src3.at[wid, ch + 4], srcs[ib], semi)
                    pltpu.async_copy(dst3.at[wid, ch + 4], dsts[ib], semi)

                @pl.when(ch + 2 < NCHUNK)
                def _():
                    ib2 = (j + 2) % 4
                    pltpu.make_async_copy(src3.at[wid, 0], srcs[ib2], semi).wait()
                    pltpu.make_async_copy(dst3.at[wid, 0], dsts[ib2], semi).wait()
                    pltpu.async_copy(table.at[srcs[ib2]], rb, semg)
            return 0
        lax.fori_loop(0, NQ, quad, 0)
        plsc.subcore_barrier()
        _drain_acc(c, row0, acc, out, stages, semr, semw)

    return k


def _make_count_seg():
    @functools.partial(
        pl.kernel,
        out_type=[jax.ShapeDtypeStruct((NC, N_PAD, FD), jnp.float32),
                  jax.ShapeDtypeStruct((NC, N_PAD, FD), jnp.float32)],
        mesh=_mk_mesh(),
        scratch_types=[
            [pltpu.VMEM((CHUNK,), jnp.int32) for _ in range(4)],
            [pltpu.VMEM((CHUNK,), jnp.int32) for _ in range(4)],
            [pltpu.VMEM((CHUNK, FD), jnp.float32) for _ in range(2)],
            [pltpu.VMEM((DR, FD), jnp.float32) for _ in range(2)],
            pltpu.VMEM_SHARED((N_PAD, FD), jnp.float32),
            pltpu.SemaphoreType.DMA,
            pltpu.SemaphoreType.DMA,
            pltpu.SemaphoreType.DMA,
            pltpu.SemaphoreType.DMA,
        ],
    )
    def k(table, src3, dst3, out, cnt_out, srcs, dsts, rows, stages, acc,
          semg, semi, semr, semw):
        c = lax.axis_index("c")
        s = lax.axis_index("s")
        wid = c * NS + s
        zero16 = jnp.zeros((16,), jnp.float32)
        one16 = jnp.ones((16,), jnp.float32)

        def zrow(i, _):
            r = i // (FD // 16)
            q = i % (FD // 16)
            stages[0][r, pl.ds(q * 16, 16)] = zero16
            return 0
        lax.fori_loop(0, DR * (FD // 16), zrow, 0)

        def orow(i, _):
            r = i // (FD // 16)
            q = i % (FD // 16)
            rows[0][r, pl.ds(q * 16, 16)] = one16
            return 0
        lax.fori_loop(0, CHUNK * (FD // 16), orow, 0)

        row0 = s * RPT

        def zcp(kk, _):
            r = pl.multiple_of(row0 + kk * DR, 8)
            pltpu.async_copy(stages[0], acc.at[pl.ds(r, DR)], semw)
            return 0
        lax.fori_loop(0, NDR, zcp, 0)

        for j in range(4):
            pltpu.async_copy(dst3.at[wid, j], dsts[j], semi)

        def zwait(kk, _):
            pltpu.make_async_copy(acc.at[pl.ds(0, DR)], stages[0], semw).wait()
            return 0
        lax.fori_loop(0, NDR, zwait, 0)
        plsc.subcore_barrier()

        def cquad(qq, _):
            q = qq * 4
            for j in range(4):
                ch = q + j

                @pl.when(ch < NCHUNK)
                def _():
                    pltpu.make_async_copy(dst3.at[wid, 0], dsts[j], semi).wait()
                    pltpu.sync_copy(rows[0], acc.at[dsts[j]], add=True)

                @pl.when(ch + 4 < NCHUNK)
                def _():
                    pltpu.async_copy(dst3.at[wid, ch + 4], dsts[j], semi)
            return 0
        lax.fori_loop(0, NQ, cquad, 0)
        plsc.subcore_barrier()

        pltpu.sync_copy(src3.at[wid, 0], srcs[0])
        pltpu.sync_copy(dst3.at[wid, 0], dsts[0])
        pltpu.sync_copy(src3.at[wid, 1], srcs[1])
        pltpu.sync_copy(dst3.at[wid, 1], dsts[1])
        pltpu.async_copy(table.at[srcs[0]], rows[0], semg)
        pltpu.async_copy(table.at[srcs[1]], rows[1], semg)
        pltpu.async_copy(src3.at[wid, 2], srcs[2], semi)
        pltpu.async_copy(dst3.at[wid, 2], dsts[2], semi)
        pltpu.async_copy(src3.at[wid, 3], srcs[3], semi)
        pltpu.async_copy(dst3.at[wid, 3], dsts[3], semi)

        _drain_acc(c, row0, acc, cnt_out, stages, semr, semw)
        lax.fori_loop(0, DR * (FD // 16), zrow, 0)
        lax.fori_loop(0, NDR, zcp, 0)
        lax.fori_loop(0, NDR, zwait, 0)
        plsc.subcore_barrier()

        def quad(qq, _):
            q = qq * 4
            for j in range(4):
                ch = q + j
                rb = rows[j % 2]
                ib = j

                @pl.when(ch < NCHUNK)
                def _():
                    pltpu.make_async_copy(table.at[pl.ds(0, CHUNK)], rb, semg).wait()
                    pltpu.sync_copy(rb, acc.at[dsts[ib]], add=True)

                @pl.when(ch + 4 < NCHUNK)
                def _():
                    pltpu.async_copy(src3.at[wid, ch + 4], srcs[ib], semi)
                    pltpu.async_copy(dst3.at[wid, ch + 4], dsts[ib], semi)

                @pl.when(ch + 2 < NCHUNK)
                def _():
                    ib2 = (j + 2) % 4
                    pltpu.make_async_copy(src3.at[wid, 0], srcs[ib2], semi).wait()
                    pltpu.make_async_copy(dst3.at[wid, 0], dsts[ib2], semi).wait()
                    pltpu.async_copy(table.at[srcs[ib2]], rb, semg)
            return 0
        lax.fori_loop(0, NQ, quad, 0)
        plsc.subcore_barrier()
        _drain_acc(c, row0, acc, out, stages, semr, semw)

    return k


def _make_count():
    @functools.partial(
        pl.kernel,
        out_type=jax.ShapeDtypeStruct((NC, N_PAD, FD), jnp.float32),
        mesh=_mk_mesh(),
        scratch_types=[
            [pltpu.VMEM((CHUNK,), jnp.int32) for _ in range(4)],
            pltpu.VMEM((CHUNK, FD), jnp.float32),
            [pltpu.VMEM((DR, FD), jnp.float32) for _ in range(2)],
            pltpu.VMEM_SHARED((N_PAD, FD), jnp.float32),
            pltpu.SemaphoreType.DMA,
            pltpu.SemaphoreType.DMA,
            pltpu.SemaphoreType.DMA,
        ],
    )
    def k(dst3, out, dsts, ones_v, stages, acc, semi, semr, semw):
        c = lax.axis_index("c")
        s = lax.axis_index("s")
        wid = c * NS + s
        zero16 = jnp.zeros((16,), jnp.float32)
        one16 = jnp.ones((16,), jnp.float32)

        def zrow(i, _):
            r = i // (FD // 16)
            q = i % (FD // 16)
            stages[0][r, pl.ds(q * 16, 16)] = zero16
            return 0
        lax.fori_loop(0, DR * (FD // 16), zrow, 0)

        def orow(i, _):
            r = i // (FD // 16)
            q = i % (FD // 16)
            ones_v[r, pl.ds(q * 16, 16)] = one16
            return 0
        lax.fori_loop(0, CHUNK * (FD // 16), orow, 0)

        row0 = s * RPT

        def zcp(kk, _):
            r = pl.multiple_of(row0 + kk * DR, 8)
            pltpu.async_copy(stages[0], acc.at[pl.ds(r, DR)], semw)
            return 0
        lax.fori_loop(0, NDR, zcp, 0)

        def zwait(kk, _):
            pltpu.make_async_copy(acc.at[pl.ds(0, DR)], stages[0], semw).wait()
            return 0
        lax.fori_loop(0, NDR, zwait, 0)
        plsc.subcore_barrier()

        for j in range(4):
            pltpu.async_copy(dst3.at[wid, j], dsts[j], semi)

        def quad(qq, _):
            q = qq * 4
            for j in range(4):
                ch = q + j

                @pl.when(ch < NCHUNK)
                def _():
                    pltpu.make_async_copy(dst3.at[wid, 0], dsts[j], semi).wait()
                    pltpu.sync_copy(ones_v, acc.at[dsts[j]], add=True)

                @pl.when(ch + 4 < NCHUNK)
                def _():
                    pltpu.async_copy(dst3.at[wid, ch + 4], dsts[j], semi)
            return 0
        lax.fori_loop(0, NQ, quad, 0)
        plsc.subcore_barrier()
        _drain_acc(c, row0, acc, out, stages, semr, semw)

    return k


_seg_sum = _make_seg_sum()
_count_seg_sum = _make_count_seg()


HALF = N_NODES // 2


def _project(x_src, x_dst, W_src, b_src, W_dst, b_dst):
    def body(xs, xd, ws, bs, wd, bd, o):
        o[0:HALF, :] = (
            jnp.dot(xs[...], ws[...], preferred_element_type=jnp.float32) + bs[...]
        )
        o[HALF:N_NODES, :] = (
            jnp.dot(xd[...], wd[...], preferred_element_type=jnp.float32) + bd[...]
        )

    return pl.pallas_call(
        body,
        out_shape=jax.ShapeDtypeStruct((N_NODES, FD), jnp.float32),
    )(x_src, x_dst, W_src, b_src.reshape(1, FD), W_dst, b_dst.reshape(1, FD))


RB = 2000


def _layer1(parts, cnts, x, W1_l, b1_l, W1_r):
    grid = (N_NODES // RB,)

    def body(p_ref, c_ref, x_ref, wl, bl, wr, hpre_ref, stat_ref):
        i = pl.program_id(0)
        psum = p_ref[0] + p_ref[1]
        cnt = c_ref[0, :, 0:1] + c_ref[1, :, 0:1]
        inv = 1.0 / jnp.maximum(cnt, 1.0)
        agg = psum * inv
        hp = (
            jnp.dot(agg, wl[...], preferred_element_type=jnp.float32)
            + bl[...]
            + jnp.dot(x_ref[...], wr[...], preferred_element_type=jnp.float32)
        )
        hpre_ref[...] = hp

        @pl.when(i == 0)
        def _():
            stat_ref[...] = jnp.zeros_like(stat_ref)

        stat_ref[0:1, :] += jnp.sum(hp, axis=0, keepdims=True)
        stat_ref[1:2, :] += jnp.sum(hp * hp, axis=0, keepdims=True)

    return pl.pallas_call(
        body,
        grid=grid,
        in_specs=[
            pl.BlockSpec((2, RB, FD), lambda i: (0, i, 0)),
            pl.BlockSpec((2, RB, FD), lambda i: (0, i, 0)),
            pl.BlockSpec((RB, FD), lambda i: (i, 0)),
            pl.BlockSpec((FD, FD), lambda i: (0, 0)),
            pl.BlockSpec((1, FD), lambda i: (0, 0)),
            pl.BlockSpec((FD, FD), lambda i: (0, 0)),
        ],
        out_specs=[
            pl.BlockSpec((RB, FD), lambda i: (i, 0)),
            pl.BlockSpec((8, FD), lambda i: (0, 0)),
        ],
        out_shape=[
            jax.ShapeDtypeStruct((N_NODES, FD), jnp.float32),
            jax.ShapeDtypeStruct((8, FD), jnp.float32),
        ],
    )(parts, cnts, x, W1_l, b1_l.reshape(1, FD), W1_r)


def _bn_relu(hpre, stats, gamma, beta):
    grid = (N_NODES // RB,)

    def body(h_ref, st_ref, g_ref, b_ref, o_ref):
        mu = st_ref[0:1, :] / N_NODES
        var = st_ref[1:2, :] / N_NODES - mu * mu
        scale = g_ref[...] * lax.rsqrt(var + 1e-5)
        shift = b_ref[...] - mu * scale
        o_ref[...] = jnp.maximum(h_ref[...] * scale + shift, 0.0)

    return pl.pallas_call(
        body,
        grid=grid,
        in_specs=[
            pl.BlockSpec((RB, FD), lambda i: (i, 0)),
            pl.BlockSpec((8, FD), lambda i: (0, 0)),
            pl.BlockSpec((1, FD), lambda i: (0, 0)),
            pl.BlockSpec((1, FD), lambda i: (0, 0)),
        ],
        out_specs=pl.BlockSpec((RB, FD), lambda i: (i, 0)),
        out_shape=jax.ShapeDtypeStruct((N_NODES, FD), jnp.float32),
    )(hpre, stats, gamma.reshape(1, FD), beta.reshape(1, FD))


def _layer2(parts, cnts, h, W2_l, b2_l, W2_r):
    grid = (N_NODES // RB,)

    def body(p_ref, c_ref, h_ref, wl, bl, wr, o_ref):
        psum = p_ref[0] + p_ref[1]
        cnt = c_ref[0, :, 0:1] + c_ref[1, :, 0:1]
        inv = 1.0 / jnp.maximum(cnt, 1.0)
        agg = psum * inv
        o_ref[...] = (
            jnp.dot(agg, wl[...], preferred_element_type=jnp.float32)
            + bl[...]
            + jnp.dot(h_ref[...], wr[...], preferred_element_type=jnp.float32)
        )

    return pl.pallas_call(
        body,
        grid=grid,
        in_specs=[
            pl.BlockSpec((2, RB, FD), lambda i: (0, i, 0)),
            pl.BlockSpec((2, RB, FD), lambda i: (0, i, 0)),
            pl.BlockSpec((RB, FD), lambda i: (i, 0)),
            pl.BlockSpec((FD, FD), lambda i: (0, 0)),
            pl.BlockSpec((1, FD), lambda i: (0, 0)),
            pl.BlockSpec((FD, FD), lambda i: (0, 0)),
        ],
        out_specs=pl.BlockSpec((RB, FD), lambda i: (i, 0)),
        out_shape=jax.ShapeDtypeStruct((N_NODES, FD), jnp.float32),
    )(parts, cnts, h, W2_l, b2_l.reshape(1, FD), W2_r)


def kernel(x_src, x_dst, edge_index, W_src, b_src, W_dst, b_dst,
           W1_l, b1_l, W1_r, W2_l, b2_l, W2_r, gamma, beta):
    src = edge_index[0].reshape(NW, NCHUNK, CHUNK)
    dst = edge_index[1].reshape(NW, NCHUNK, CHUNK)
    x = _project(x_src, x_dst, W_src, b_src, W_dst, b_dst)
    parts1, cnts = _count_seg_sum(x, src, dst)
    hpre, stats = _layer1(parts1, cnts, x, W1_l, b1_l, W1_r)
    h = _bn_relu(hpre, stats, gamma, beta)
    parts2 = _seg_sum(h, src, dst)
    out = _layer2(parts2, cnts, h, W2_l, b2_l, W2_r)
    return out
